# baseline (speedup 1.0000x reference)
import jax
import jax.numpy as jnp
from jax.experimental import pallas as pl
from jax.experimental.pallas import tpu as pltpu

_BM = 1024
_NBUF = 4


def _router_body(x_hbm, wt_ref, idx_ref, pw_ref, xbuf, sems):
    m_tot = x_hbm.shape[0]
    e_dim = wt_ref.shape[1]
    nsteps = m_tot // _BM
    wt = wt_ref[...]
    ones = jnp.ones((e_dim, e_dim), dtype=jnp.float32)
    lane = jax.lax.broadcasted_iota(jnp.int32, (1, e_dim), 1)
    w2 = jax.lax.bitcast_convert_type((127 - lane) << 23, jnp.float32)

    def slab_copy(step, slot):
        return pltpu.make_async_copy(
            x_hbm.at[pl.ds(step * _BM, _BM), :], xbuf.at[slot], sems.at[slot])

    for b in range(min(_NBUF - 1, nsteps)):
        slab_copy(b, b).start()

    rows = _BM // 128
    for i in range(nsteps):
        nxt = i + _NBUF - 1
        if nxt < nsteps:
            slab_copy(nxt, nxt % _NBUF).start()
        slot = i % _NBUF
        slab_copy(i, slot).wait()
        xs = xbuf[slot]
        logits = jnp.dot(xs, wt, preferred_element_type=jnp.float32)
        m = jnp.max(logits, axis=-1, keepdims=True)
        e = jnp.exp(logits - m)
        s = jax.lax.dot_general(e, ones, (((1,), (0,)), ((), ())),
                                preferred_element_type=jnp.float32)
        pw_ref[pl.ds(i * _BM, _BM), :] = e * (1.0 / s)
        v = jnp.where(logits == m, w2, 0.0)
        t = jax.lax.dot_general(v, ones, (((1,), (0,)), ((), ())),
                                preferred_element_type=jnp.float32)
        bits = jax.lax.bitcast_convert_type(t[:, :1], jnp.int32)
        idx = jnp.maximum(127 - (bits >> 23), 0)
        idx_ref[pl.ds(i * rows, rows), :] = idx.reshape((rows, 128))


def kernel(x, W):
    M, K = x.shape
    E = W.shape[0]
    wt = W.T
    idx, pw = pl.pallas_call(
        _router_body,
        in_specs=[
            pl.BlockSpec(memory_space=pltpu.HBM),
            pl.BlockSpec(memory_space=pltpu.VMEM),
        ],
        out_specs=[
            pl.BlockSpec(memory_space=pltpu.VMEM),
            pl.BlockSpec(memory_space=pltpu.VMEM),
        ],
        out_shape=[
            jax.ShapeDtypeStruct((M // 128, 128), jnp.int32),
            jax.ShapeDtypeStruct((M, E), jnp.float32),
        ],
        scratch_shapes=[
            pltpu.VMEM((_NBUF, _BM, K), jnp.float32),
            pltpu.SemaphoreType.DMA((_NBUF,)),
        ],
    )(x, wt)
    return idx.reshape((M,)), pw


# speedup vs baseline: 1.0409x; 1.0409x over previous
import jax
import jax.numpy as jnp
from jax.experimental import pallas as pl
from jax.experimental.pallas import tpu as pltpu

_BM = 2048
_NBUF = 3


def _router_body(x_hbm, wt_ref, idx_ref, pw_hbm, xbuf, pwbuf, sems, osems):
    m_tot = x_hbm.shape[0]
    e_dim = wt_ref.shape[1]
    nsteps = m_tot // _BM
    rows = _BM // 128
    wt = wt_ref[...]
    ones = jnp.ones((e_dim, e_dim), dtype=jnp.float32)
    lane = jax.lax.broadcasted_iota(jnp.int32, (1, e_dim), 1)
    w2 = jax.lax.bitcast_convert_type((127 - lane) << 23, jnp.float32)

    def slab_copy(step, slot):
        return pltpu.make_async_copy(
            x_hbm.at[pl.ds(step * _BM, _BM), :], xbuf.at[slot], sems.at[slot])

    def pw_copy(step, slot):
        return pltpu.make_async_copy(
            pwbuf.at[slot], pw_hbm.at[pl.ds(step * _BM, _BM), :],
            osems.at[slot])

    for b in range(min(_NBUF - 1, nsteps)):
        slab_copy(b, b).start()

    def step_fn(i, carry):
        nxt = i + _NBUF - 1

        @pl.when(nxt < nsteps)
        def _():
            slab_copy(nxt, nxt % _NBUF).start()

        slot = jax.lax.rem(i, _NBUF)
        slab_copy(i, slot).wait()
        xs = xbuf[slot]
        logits = jnp.dot(xs, wt, preferred_element_type=jnp.float32)
        m = jnp.max(logits, axis=-1, keepdims=True)
        e = jnp.exp(logits - m)
        s = jax.lax.dot_general(e, ones, (((1,), (0,)), ((), ())),
                                preferred_element_type=jnp.float32)
        oslot = jax.lax.rem(i, 2)

        @pl.when(i >= 2)
        def _():
            pw_copy(i - 2, oslot).wait()

        pwbuf[oslot] = e * (1.0 / s)
        pw_copy(i, oslot).start()
        v = jnp.where(logits == m, w2, 0.0)
        t = jax.lax.dot_general(v, ones, (((1,), (0,)), ((), ())),
                                preferred_element_type=jnp.float32)
        bits = jax.lax.bitcast_convert_type(t[:, :1], jnp.int32)
        idx = jnp.maximum(127 - (bits >> 23), 0)
        idx_ref[pl.ds(i * rows, rows), :] = idx.reshape((rows, 128))
        return carry

    jax.lax.fori_loop(0, nsteps, step_fn, 0)
    for tail in range(max(0, nsteps - 2), nsteps):
        pw_copy(tail, tail % 2).wait()


def kernel(x, W):
    M, K = x.shape
    E = W.shape[0]
    wt = W.T
    idx, pw = pl.pallas_call(
        _router_body,
        in_specs=[
            pl.BlockSpec(memory_space=pltpu.HBM),
            pl.BlockSpec(memory_space=pltpu.VMEM),
        ],
        out_specs=[
            pl.BlockSpec(memory_space=pltpu.VMEM),
            pl.BlockSpec(memory_space=pltpu.HBM),
        ],
        out_shape=[
            jax.ShapeDtypeStruct((M // 128, 128), jnp.int32),
            jax.ShapeDtypeStruct((M, E), jnp.float32),
        ],
        scratch_shapes=[
            pltpu.VMEM((_NBUF, _BM, K), jnp.float32),
            pltpu.VMEM((2, _BM, E), jnp.float32),
            pltpu.SemaphoreType.DMA((_NBUF,)),
            pltpu.SemaphoreType.DMA((2,)),
        ],
    )(x, wt)
    return idx.reshape((M,)), pw


# speedup vs baseline: 1.0561x; 1.0146x over previous
import jax
import jax.numpy as jnp
from jax.experimental import pallas as pl
from jax.experimental.pallas import tpu as pltpu

_BM = 1024
_HM = _BM // 2
_NBUF = 4


def _router_body(x_hbm, wt_ref, idx_ref, pw_ref, xbuf, sems_a, sems_b):
    m_tot = x_hbm.shape[0]
    e_dim = wt_ref.shape[1]
    nsteps = m_tot // _BM
    rows = _BM // 128
    wt = wt_ref[...]
    ones = jnp.ones((e_dim, e_dim), dtype=jnp.float32)
    lane = jax.lax.broadcasted_iota(jnp.int32, (1, e_dim), 1)
    w2 = jax.lax.bitcast_convert_type((127 - lane) << 23, jnp.float32)

    def copy_a(step, slot):
        return pltpu.make_async_copy(
            x_hbm.at[pl.ds(step * _BM, _HM), :],
            xbuf.at[slot, pl.ds(0, _HM), :], sems_a.at[slot])

    def copy_b(step, slot):
        return pltpu.make_async_copy(
            x_hbm.at[pl.ds(step * _BM + _HM, _HM), :],
            xbuf.at[slot, pl.ds(_HM, _HM), :], sems_b.at[slot])

    for b in range(min(_NBUF - 1, nsteps)):
        copy_a(b, b).start()
        copy_b(b, b).start()

    def step_fn(i, carry):
        nxt = i + _NBUF - 1

        @pl.when(nxt < nsteps)
        def _():
            copy_a(nxt, nxt % _NBUF).start()
            copy_b(nxt, nxt % _NBUF).start()

        slot = jax.lax.rem(i, _NBUF)
        copy_a(i, slot).wait()
        copy_b(i, slot).wait()
        xs = xbuf[slot]
        logits = jnp.dot(xs, wt, preferred_element_type=jnp.float32)
        m = jnp.max(logits, axis=-1, keepdims=True)
        e = jnp.exp(logits - m)
        s = jax.lax.dot_general(e, ones, (((1,), (0,)), ((), ())),
                                preferred_element_type=jnp.float32)
        pw_ref[pl.ds(i * _BM, _BM), :] = e * (1.0 / s)
        v = jnp.where(logits == m, w2, 0.0)
        t = jax.lax.dot_general(v, ones, (((1,), (0,)), ((), ())),
                                preferred_element_type=jnp.float32)
        bits = jax.lax.bitcast_convert_type(t[:, :1], jnp.int32)
        idx = jnp.maximum(127 - (bits >> 23), 0)
        idx_ref[pl.ds(i * rows, rows), :] = idx.reshape((rows, 128))
        return carry

    jax.lax.fori_loop(0, nsteps, step_fn, 0)


def kernel(x, W):
    M, K = x.shape
    E = W.shape[0]
    wt = W.T
    idx, pw = pl.pallas_call(
        _router_body,
        in_specs=[
            pl.BlockSpec(memory_space=pltpu.HBM),
            pl.BlockSpec(memory_space=pltpu.VMEM),
        ],
        out_specs=[
            pl.BlockSpec(memory_space=pltpu.VMEM),
            pl.BlockSpec(memory_space=pltpu.VMEM),
        ],
        out_shape=[
            jax.ShapeDtypeStruct((M // 128, 128), jnp.int32),
            jax.ShapeDtypeStruct((M, E), jnp.float32),
        ],
        scratch_shapes=[
            pltpu.VMEM((_NBUF, _BM, K), jnp.float32),
            pltpu.SemaphoreType.DMA((_NBUF,)),
            pltpu.SemaphoreType.DMA((_NBUF,)),
        ],
    )(x, wt)
    return idx.reshape((M,)), pw


# speedup vs baseline: 1.0641x; 1.0076x over previous
import jax
import jax.numpy as jnp
from jax.experimental import pallas as pl
from jax.experimental.pallas import tpu as pltpu

_BM = 1024
_NBUF = 4


def _router_body(x_hbm, wt_ref, idx_ref, pw_hbm, xbuf, pwbuf, sems, osems):
    m_tot = x_hbm.shape[0]
    e_dim = wt_ref.shape[1]
    nsteps = m_tot // _BM
    rows = _BM // 128
    wt = wt_ref[...]
    ones = jnp.ones((e_dim, e_dim), dtype=jnp.float32)
    lane = jax.lax.broadcasted_iota(jnp.int32, (1, e_dim), 1)
    w2 = jax.lax.bitcast_convert_type((127 - lane) << 23, jnp.float32)

    def slab_copy(step, slot):
        return pltpu.make_async_copy(
            x_hbm.at[pl.ds(step * _BM, _BM), :], xbuf.at[slot], sems.at[slot])

    def pw_copy(step, slot):
        return pltpu.make_async_copy(
            pwbuf.at[slot], pw_hbm.at[pl.ds(step * _BM, _BM), :],
            osems.at[slot])

    for b in range(min(_NBUF - 1, nsteps)):
        slab_copy(b, b).start()

    def step_fn(i, carry):
        nxt = i + _NBUF - 1

        @pl.when(nxt < nsteps)
        def _():
            slab_copy(nxt, nxt % _NBUF).start()

        slot = jax.lax.rem(i, _NBUF)
        slab_copy(i, slot).wait()
        xs = xbuf[slot]
        logits = jnp.dot(xs, wt, preferred_element_type=jnp.float32)
        m = jnp.max(logits, axis=-1, keepdims=True)
        e = jnp.exp(logits - m)
        s = jax.lax.dot_general(e, ones, (((1,), (0,)), ((), ())),
                                preferred_element_type=jnp.float32)
        oslot = jax.lax.rem(i, 2)

        @pl.when(i >= 2)
        def _():
            pw_copy(i - 2, oslot).wait()

        pwbuf[oslot] = e * (1.0 / s)
        pw_copy(i, oslot).start()
        v = jnp.where(logits == m, w2, 0.0)
        t = jax.lax.dot_general(v, ones, (((1,), (0,)), ((), ())),
                                preferred_element_type=jnp.float32)
        bits = jax.lax.bitcast_convert_type(t[:, :1], jnp.int32)
        idx = jnp.maximum(127 - (bits >> 23), 0)
        idx_ref[pl.ds(i * rows, rows), :] = idx.reshape((rows, 128))
        return carry

    jax.lax.fori_loop(0, nsteps, step_fn, 0)
    for tail in range(max(0, nsteps - 2), nsteps):
        pw_copy(tail, tail % 2).wait()


def kernel(x, W):
    M, K = x.shape
    E = W.shape[0]
    wt = W.T
    idx, pw = pl.pallas_call(
        _router_body,
        in_specs=[
            pl.BlockSpec(memory_space=pltpu.HBM),
            pl.BlockSpec(memory_space=pltpu.VMEM),
        ],
        out_specs=[
            pl.BlockSpec(memory_space=pltpu.VMEM),
            pl.BlockSpec(memory_space=pltpu.HBM),
        ],
        out_shape=[
            jax.ShapeDtypeStruct((M // 128, 128), jnp.int32),
            jax.ShapeDtypeStruct((M, E), jnp.float32),
        ],
        scratch_shapes=[
            pltpu.VMEM((_NBUF, _BM, K), jnp.float32),
            pltpu.VMEM((2, _BM, E), jnp.float32),
            pltpu.SemaphoreType.DMA((_NBUF,)),
            pltpu.SemaphoreType.DMA((2,)),
        ],
    )(x, wt)
    return idx.reshape((M,)), pw
